# baseline (speedup 1.0000x reference)
import functools
import math

import numpy as np
import jax
import jax.numpy as jnp
from jax import lax
from jax.experimental import pallas as pl
from jax.experimental.pallas import tpu as pltpu
from jax.experimental.pallas import tpu_sc as plsc

B = 4
H = 224
NSEG = 196
NPAD = 208
NW = 32
WPI = 8
PROC_ROWS = H // WPI
WIN_ROWS = 32
SQRT3 = math.sqrt(3.0)
SQRT96 = math.sqrt(96.0)


def _resize_matrix_np(in_n, out_n):
    ys = np.linspace(0.0, in_n - 1.0, out_n, dtype=np.float32)
    y0 = np.floor(ys).astype(np.int32)
    y1 = np.clip(y0 + 1, 0, in_n - 1)
    wy = (ys - y0.astype(np.float32)).astype(np.float32)
    m = np.zeros((out_n, in_n), dtype=np.float32)
    m[np.arange(out_n), y0] += 1.0 - wy
    m[np.arange(out_n), y1] += wy
    return m


_RY = _resize_matrix_np(56, 224)


def _stage_a_body(in_ref, feat_ref, ry_ref, ryt_ref, o1_ref, o2_ref):
    ry = ry_ref[...]
    ryt = ryt_ref[...]
    for b in range(B):
        o1_ref[b] = jnp.sum(in_ref[b], axis=0)
        fb = jnp.sum(feat_ref[b], axis=2)
        t = lax.dot(ry, fb, precision=lax.Precision.HIGHEST,
                    preferred_element_type=jnp.float32)
        o2_ref[b] = lax.dot(t, ryt, precision=lax.Precision.HIGHEST,
                            preferred_element_type=jnp.float32)


_stage_a = pl.pallas_call(
    _stage_a_body,
    out_shape=[
        jax.ShapeDtypeStruct((B, H, H), jnp.float32),
        jax.ShapeDtypeStruct((B, H, H), jnp.float32),
    ],
)


_sc_mesh = plsc.VectorSubcoreMesh(core_axis_name="c", subcore_axis_name="s")


@functools.partial(
    pl.kernel,
    mesh=_sc_mesh,
    compiler_params=pltpu.CompilerParams(needs_layout_passes=False),
    out_type=jax.ShapeDtypeStruct((NW, 3 * NPAD), jnp.float32),
    scratch_types=(
        pltpu.VMEM((WIN_ROWS, H), jnp.int32),
        pltpu.VMEM((WIN_ROWS, H), jnp.float32),
        pltpu.VMEM((WIN_ROWS, H), jnp.float32),
        pltpu.VMEM((3 * NPAD,), jnp.float32),
        pltpu.SemaphoreType.DMA,
    ),
)
def _sc_segment_sums(seg_hbm, v1_hbm, v2_hbm, out_hbm,
                     seg_s, v1_s, v2_s, acc, sem):
    wid = lax.axis_index("s") * 2 + lax.axis_index("c")
    b = wid // WPI
    chunk = wid % WPI
    d = 4 * (chunk % 2)
    r0 = pl.multiple_of(PROC_ROWS * chunk - d, 8)
    rs = pl.ds(r0, WIN_ROWS)
    cps = [
        pltpu.async_copy(seg_hbm.at[b, 0, rs, :], seg_s, sem),
        pltpu.async_copy(v1_hbm.at[b, rs, :], v1_s, sem),
        pltpu.async_copy(v2_hbm.at[b, rs, :], v2_s, sem),
    ]

    zeros16 = jnp.zeros((16,), jnp.float32)

    def zero_body(i, c):
        acc[pl.ds(i * 16, 16)] = zeros16
        return c

    lax.fori_loop(0, 3 * NPAD // 16, zero_body, 0)
    for cp in cps:
        cp.wait()

    ones16 = jnp.ones((16,), jnp.float32)
    off1 = jnp.full((16,), NPAD, jnp.int32)
    off2 = jnp.full((16,), 2 * NPAD, jnp.int32)
    GRP = 7

    def row_body(r, c):
        rr = r + d
        for g in range(H // 16 // GRP):
            sls = [pl.ds((g * GRP + j) * 16, 16) for j in range(GRP)]
            idxs = [seg_s[rr, sl] for sl in sls]
            v1l = [v1_s[rr, sl] for sl in sls]
            v2l = [v2_s[rr, sl] for sl in sls]
            for j in range(GRP):
                plsc.addupdate_scatter(acc, [idxs[j]], v1l[j])
                plsc.addupdate_scatter(acc, [idxs[j] + off1], v2l[j])
                plsc.addupdate_scatter(acc, [idxs[j] + off2], ones16)
        return c

    lax.fori_loop(0, PROC_ROWS, row_body, 0)

    pltpu.sync_copy(acc, out_hbm.at[wid])


def _stage_c_body(p_ref, num_ref, out_ref):
    pall = p_ref[...]
    num0 = num_ref[0]
    ri = lax.broadcasted_iota(jnp.int32, (NPAD, NPAD), 0)
    ci = lax.broadcasted_iota(jnp.int32, (NPAD, NPAD), 1)
    valid = (ri < num0) & (ci < num0)
    row_iota = lax.broadcasted_iota(jnp.int32, (1, NPAD), 1)

    m1_rows = []
    m2_rows = []
    for b in range(B):
        grp = jnp.sum(lax.slice(pall, (WPI * b, 0), (WPI * b + WPI, 3 * NPAD)),
                      axis=0, keepdims=True)
        s1r = lax.slice(grp, (0, 0), (1, NPAD))
        s2r = lax.slice(grp, (0, NPAD), (1, 2 * NPAD))
        scr = lax.slice(grp, (0, 2 * NPAD), (1, 3 * NPAD))
        okr = (scr > 0) & (row_iota < num_ref[b])
        m1_rows.append(jnp.where(okr, s1r / (jnp.maximum(scr, 1.0) * 3.0), 0.0))
        m2_rows.append(jnp.where(okr, s2r / (jnp.maximum(scr, 1.0) * 96.0), 0.0))

    m1 = lax.concatenate(m1_rows, 0)
    m2 = lax.concatenate(m2_rows, 0)
    m1t = jnp.transpose(m1)
    m2t = jnp.transpose(m2)

    total = jnp.zeros((), jnp.float32)
    for b in range(B):
        m1r = lax.slice(m1, (b, 0), (b + 1, NPAD))
        m2r = lax.slice(m2, (b, 0), (b + 1, NPAD))
        m1c = lax.slice(m1t, (0, b), (NPAD, b + 1))
        m2c = lax.slice(m2t, (0, b), (NPAD, b + 1))
        d1 = m1c - m1r
        d2 = m2c - m2r
        n1 = jnp.sqrt(jnp.maximum(3.0 * d1 * d1, 1e-24)) / SQRT3
        n2 = jnp.sqrt(jnp.maximum(96.0 * d2 * d2, 1e-24)) / SQRT96
        total = total + jnp.sum(jnp.where(valid, jnp.abs(n2 - n1), 0.0))

    count = (4 * num0 * num0).astype(jnp.float32)
    out_ref[...] = (total / count) * jnp.ones((1, 1), jnp.float32)


_stage_c = pl.pallas_call(
    _stage_c_body,
    in_specs=[pl.BlockSpec(memory_space=pltpu.VMEM),
              pl.BlockSpec(memory_space=pltpu.SMEM)],
    out_shape=jax.ShapeDtypeStruct((1, 1), jnp.float32),
)


def kernel(input, feature, sp, num):
    ry = jnp.asarray(_RY)
    ryt = jnp.asarray(_RY.T)
    xsum1, xsum2 = _stage_a(input, jnp.transpose(feature, (0, 2, 3, 1)), ry, ryt)
    partials = _sc_segment_sums(sp.astype(jnp.int32), xsum1, xsum2)
    out = _stage_c(partials, num.astype(jnp.int32))
    return out[0, 0]
